# baseline (speedup 1.0000x reference)
import functools

import jax
import jax.numpy as jnp
from jax import lax
from jax.experimental import pallas as pl
from jax.experimental.pallas import tpu as pltpu
from jax.experimental.pallas import tpu_sc as plsc

VOCAB = 100000
DIM = 128
BATCH = 16384
X_MAX = 100.0
ALPHA = 0.75

NC = 2
NS = 16
L = 16
NW = NC * NS
BPW = BATCH // NW
CH = 128
NCHUNK = BPW // CH

_mesh = plsc.VectorSubcoreMesh(core_axis_name="c", subcore_axis_name="s")

_GATHER_DNUMS = lax.GatherDimensionNumbers(
    offset_dims=(), collapsed_slice_dims=(0,), start_index_map=(0,))


def _lane_perm(v, perm):
    return lax.gather(v, perm[:, None], _GATHER_DNUMS, (1,),
                      mode=lax.GatherScatterMode.PROMISE_IN_BOUNDS)


def _stage_consts():
    lane = lax.iota(jnp.int32, L)
    out = []
    for lvl in range(4):
        s = 1 << lvl
        out.append((jnp.bitwise_xor(lane, s),
                    jnp.bitwise_and(lane, s) == 0))
    return out


def _combine(a, b, perm, m):
    ar = _lane_perm(a, perm)
    br = _lane_perm(b, perm)
    return jnp.where(m, a, br) + jnp.where(m, ar, b)


_LN2 = 0.6931471805599453
_LN_XMAX = 4.605170185988092


def _ln(x):
    e = jnp.zeros((L,), jnp.float32)
    m = x
    for kexp in (16, 8, 4, 2, 1):
        thr = float(2 ** kexp)
        cond = m >= thr
        m = jnp.where(cond, m * (1.0 / thr), m)
        e = e + jnp.where(cond, float(kexp), 0.0)
    s = (m - 1.0) / (m + 1.0)
    s2 = s * s
    p = 2.0 * s * (1.0 + s2 * (1.0 / 3.0 + s2 * (0.2 + s2 * (1.0 / 7.0
                   + s2 * (1.0 / 9.0)))))
    return p + e * _LN2


def _lane_sum_16(vecs, stages):
    lvl = 0
    while len(vecs) > 1:
        perm, m = stages[lvl]
        vecs = [_combine(vecs[k], vecs[k + 1], perm, m)
                for k in range(0, len(vecs), 2)]
        lvl += 1
    return vecs[0]


@functools.partial(
    pl.kernel,
    out_type=jax.ShapeDtypeStruct((NW, L), jnp.float32),
    mesh=_mesh,
    scratch_types=(
        [pltpu.VMEM((BPW,), jnp.int32)] * 2
        + [pltpu.VMEM((BPW,), jnp.float32)]
        + [pltpu.VMEM((2 * CH, DIM), jnp.float32)] * 2
        + [pltpu.VMEM((2 * CH,), jnp.float32)] * 2
        + [pltpu.VMEM((L * L,), jnp.float32)]
        + [pltpu.VMEM((L,), jnp.float32)]
        + [pltpu.SemaphoreType.DMA] * 2
    ),
)
def _glove_sc(i_hbm, j_hbm, x_hbm, w_hbm, wt_hbm, b_hbm, bt_hbm,
              out_hbm, ii_all, jj_all, xv_all, wi2, wj2,
              bi2, bj2, dmat, accbuf, sem0, sem1):
    wid = lax.axis_index("s") * NC + lax.axis_index("c")
    base = wid * BPW
    stages = _stage_consts()
    pltpu.sync_copy(i_hbm.at[pl.ds(base, BPW)], ii_all)
    pltpu.sync_copy(j_hbm.at[pl.ds(base, BPW)], jj_all)
    pltpu.sync_copy(x_hbm.at[pl.ds(base, BPW)], xv_all)
    sems = (sem0, sem1)

    def issue(loc, slot):
        soff = slot * CH
        iref = ii_all.at[pl.ds(loc, CH)]
        jref = jj_all.at[pl.ds(loc, CH)]
        sem = sems[slot]
        pltpu.async_copy(w_hbm.at[iref], wi2.at[pl.ds(soff, CH), :], sem)
        pltpu.async_copy(wt_hbm.at[jref], wj2.at[pl.ds(soff, CH), :], sem)
        pltpu.async_copy(b_hbm.at[iref], bi2.at[pl.ds(soff, CH)], sem)
        pltpu.async_copy(bt_hbm.at[jref], bj2.at[pl.ds(soff, CH)], sem)

    def drain(slot):
        soff = slot * CH
        iref = ii_all.at[pl.ds(0, CH)]
        jref = jj_all.at[pl.ds(0, CH)]
        sem = sems[slot]
        pltpu.make_async_copy(w_hbm.at[iref], wi2.at[pl.ds(soff, CH), :],
                              sem).wait()
        pltpu.make_async_copy(wt_hbm.at[jref], wj2.at[pl.ds(soff, CH), :],
                              sem).wait()
        pltpu.make_async_copy(b_hbm.at[iref], bi2.at[pl.ds(soff, CH)],
                              sem).wait()
        pltpu.make_async_copy(bt_hbm.at[jref], bj2.at[pl.ds(soff, CH)],
                              sem).wait()

    issue(0, 0)

    def chunk_body(ch, acc):
        par = jnp.bitwise_and(ch, 1)
        loc = ch * CH
        soff = par * CH

        @pl.when(par == 0)
        def _():
            drain(0)

        @pl.when(par == 1)
        def _():
            drain(1)

        @pl.when(ch + 1 < NCHUNK)
        def _():
            @pl.when(par == 0)
            def _():
                issue(loc + CH, 1)

            @pl.when(par == 1)
            def _():
                issue(loc + CH, 0)

        def grp_body(g, carry):
            def one_row(r):
                d0 = wi2[r, pl.ds(0, L)] * wj2[r, pl.ds(0, L)]
                d1 = wi2[r, pl.ds(L, L)] * wj2[r, pl.ds(L, L)]
                d2 = wi2[r, pl.ds(2 * L, L)] * wj2[r, pl.ds(2 * L, L)]
                d3 = wi2[r, pl.ds(3 * L, L)] * wj2[r, pl.ds(3 * L, L)]
                for k in range(4, DIM // L):
                    q = k % 4
                    p = wi2[r, pl.ds(k * L, L)] * wj2[r, pl.ds(k * L, L)]
                    if q == 0:
                        d0 = d0 + p
                    elif q == 1:
                        d1 = d1 + p
                    elif q == 2:
                        d2 = d2 + p
                    else:
                        d3 = d3 + p
                return (d0 + d1) + (d2 + d3)

            def quad_body(q16, t):
                r = soff + g * L + 4 * q16
                da = _combine(one_row(r), one_row(r + 1), *stages[0])
                db = _combine(one_row(r + 2), one_row(r + 3), *stages[0])
                dmat[pl.ds(q16 * L, L)] = _combine(da, db, *stages[1])
                return t

            lax.fori_loop(0, L // 4, quad_body, 0, unroll=2)
            vecs = [dmat[pl.ds(r * L, L)] for r in range(L // 4)]
            lvl = 2
            while len(vecs) > 1:
                perm, m = stages[lvl]
                vecs = [_combine(vecs[k], vecs[k + 1], perm, m)
                        for k in range(0, len(vecs), 2)]
                lvl += 1
            dotv = vecs[0]
            sl = pl.ds(soff + g * L, L)
            sg = pl.ds(loc + g * L, L)
            lnx = _ln(xv_all[sg])
            wf = jnp.minimum(jnp.exp((lnx - _LN_XMAX) * ALPHA), 1.0)
            diff = dotv + bi2[sl] + bj2[sl] - lnx
            return carry + wf * diff * diff

        return lax.fori_loop(0, CH // L, grp_body, acc, unroll=1)

    acc = lax.fori_loop(0, NCHUNK, chunk_body, jnp.zeros((L,), jnp.float32),
                        unroll=1)
    accbuf[...] = acc
    pltpu.sync_copy(accbuf, out_hbm.at[wid])


def kernel(i, j, x, W, W_tilde, bias, bias_tilde):
    parts = _glove_sc(i.astype(jnp.int32), j.astype(jnp.int32),
                      x.astype(jnp.float32), W, W_tilde, bias, bias_tilde)
    return jnp.sum(parts) / BATCH


# speedup vs baseline: 1.0027x; 1.0027x over previous
import functools

import jax
import jax.numpy as jnp
from jax import lax
from jax.experimental import pallas as pl
from jax.experimental.pallas import tpu as pltpu
from jax.experimental.pallas import tpu_sc as plsc

VOCAB = 100000
DIM = 128
BATCH = 16384
X_MAX = 100.0
ALPHA = 0.75

NC = 2
NS = 16
L = 16
NW = NC * NS
BPW = BATCH // NW
CH = 128
NCHUNK = BPW // CH

_mesh = plsc.VectorSubcoreMesh(core_axis_name="c", subcore_axis_name="s")

_GATHER_DNUMS = lax.GatherDimensionNumbers(
    offset_dims=(), collapsed_slice_dims=(0,), start_index_map=(0,))


def _lane_perm(v, perm):
    return lax.gather(v, perm[:, None], _GATHER_DNUMS, (1,),
                      mode=lax.GatherScatterMode.PROMISE_IN_BOUNDS)


def _stage_consts():
    lane = lax.iota(jnp.int32, L)
    out = []
    for lvl in range(4):
        s = 1 << lvl
        out.append((jnp.bitwise_xor(lane, s),
                    jnp.bitwise_and(lane, s) == 0))
    return out


def _combine(a, b, perm, m):
    ar = _lane_perm(a, perm)
    br = _lane_perm(b, perm)
    return jnp.where(m, a, br) + jnp.where(m, ar, b)


_LN2 = 0.6931471805599453
_LN_XMAX = 4.605170185988092


def _ln(x):
    e = jnp.zeros((L,), jnp.float32)
    m = x
    for kexp in (16, 8, 4, 2, 1):
        thr = float(2 ** kexp)
        cond = m >= thr
        m = jnp.where(cond, m * (1.0 / thr), m)
        e = e + jnp.where(cond, float(kexp), 0.0)
    s = (m - 1.0) / (m + 1.0)
    s2 = s * s
    p = 2.0 * s * (1.0 + s2 * (1.0 / 3.0 + s2 * (0.2 + s2 * (1.0 / 7.0
                   + s2 * (1.0 / 9.0)))))
    return p + e * _LN2


@functools.partial(
    pl.kernel,
    out_type=jax.ShapeDtypeStruct((NW, L), jnp.float32),
    mesh=_mesh,
    scratch_types=(
        [pltpu.VMEM((BPW,), jnp.int32)] * 2
        + [pltpu.VMEM((BPW,), jnp.float32)]
        + [pltpu.VMEM((2 * CH, DIM), jnp.float32)] * 2
        + [pltpu.VMEM((2 * CH,), jnp.float32)] * 2
        + [pltpu.VMEM((L * L,), jnp.float32)]
        + [pltpu.VMEM((L,), jnp.float32)]
        + [pltpu.SemaphoreType.DMA] * 2
    ),
)
def _glove_sc(i_hbm, j_hbm, x_hbm, w_hbm, wt_hbm, b_hbm, bt_hbm,
              out_hbm, ii_all, jj_all, xv_all, wi2, wj2,
              bi2, bj2, dmat, accbuf, sem0, sem1):
    wid = lax.axis_index("s") * NC + lax.axis_index("c")
    base = wid * BPW
    stages = _stage_consts()
    pltpu.sync_copy(i_hbm.at[pl.ds(base, BPW)], ii_all)
    pltpu.sync_copy(j_hbm.at[pl.ds(base, BPW)], jj_all)
    pltpu.sync_copy(x_hbm.at[pl.ds(base, BPW)], xv_all)
    sems = (sem0, sem1)

    def issue(loc, slot):
        soff = slot * CH
        iref = ii_all.at[pl.ds(loc, CH)]
        jref = jj_all.at[pl.ds(loc, CH)]
        sem = sems[slot]
        pltpu.async_copy(w_hbm.at[iref], wi2.at[pl.ds(soff, CH), :], sem)
        pltpu.async_copy(wt_hbm.at[jref], wj2.at[pl.ds(soff, CH), :], sem)
        pltpu.async_copy(b_hbm.at[iref], bi2.at[pl.ds(soff, CH)], sem)
        pltpu.async_copy(bt_hbm.at[jref], bj2.at[pl.ds(soff, CH)], sem)

    def drain(slot):
        soff = slot * CH
        iref = ii_all.at[pl.ds(0, CH)]
        jref = jj_all.at[pl.ds(0, CH)]
        sem = sems[slot]
        pltpu.make_async_copy(w_hbm.at[iref], wi2.at[pl.ds(soff, CH), :],
                              sem).wait()
        pltpu.make_async_copy(wt_hbm.at[jref], wj2.at[pl.ds(soff, CH), :],
                              sem).wait()
        pltpu.make_async_copy(b_hbm.at[iref], bi2.at[pl.ds(soff, CH)],
                              sem).wait()
        pltpu.make_async_copy(bt_hbm.at[jref], bj2.at[pl.ds(soff, CH)],
                              sem).wait()

    issue(0, 0)

    def chunk_body(ch, acc):
        par = jnp.bitwise_and(ch, 1)
        loc = ch * CH
        soff = par * CH

        @pl.when(par == 0)
        def _():
            drain(0)

        @pl.when(par == 1)
        def _():
            drain(1)

        @pl.when(ch + 1 < NCHUNK)
        def _():
            @pl.when(par == 0)
            def _():
                issue(loc + CH, 1)

            @pl.when(par == 1)
            def _():
                issue(loc + CH, 0)

        def grp_body(g, carry):
            def one_row(r):
                d0 = wi2[r, pl.ds(0, L)] * wj2[r, pl.ds(0, L)]
                d1 = wi2[r, pl.ds(L, L)] * wj2[r, pl.ds(L, L)]
                d2 = wi2[r, pl.ds(2 * L, L)] * wj2[r, pl.ds(2 * L, L)]
                d3 = wi2[r, pl.ds(3 * L, L)] * wj2[r, pl.ds(3 * L, L)]
                for k in range(4, DIM // L):
                    q = k % 4
                    p = wi2[r, pl.ds(k * L, L)] * wj2[r, pl.ds(k * L, L)]
                    if q == 0:
                        d0 = d0 + p
                    elif q == 1:
                        d1 = d1 + p
                    elif q == 2:
                        d2 = d2 + p
                    else:
                        d3 = d3 + p
                return (d0 + d1) + (d2 + d3)

            def quad_body(q16, t):
                r = soff + g * L + 4 * q16
                da = _combine(one_row(r), one_row(r + 1), *stages[0])
                db = _combine(one_row(r + 2), one_row(r + 3), *stages[0])
                dmat[pl.ds(q16 * L, L)] = _combine(da, db, *stages[1])
                return t

            lax.fori_loop(0, L // 4, quad_body, 0, unroll=2)
            vecs = [dmat[pl.ds(r * L, L)] for r in range(L // 4)]
            lvl = 2
            while len(vecs) > 1:
                perm, m = stages[lvl]
                vecs = [_combine(vecs[k], vecs[k + 1], perm, m)
                        for k in range(0, len(vecs), 2)]
                lvl += 1
            dotv = vecs[0]
            sl = pl.ds(soff + g * L, L)
            sg = pl.ds(loc + g * L, L)
            lnx = _ln(xv_all[sg])
            wf = jnp.minimum(jnp.exp((lnx - _LN_XMAX) * ALPHA), 1.0)
            diff = dotv + bi2[sl] + bj2[sl] - lnx
            return carry + wf * diff * diff

        return lax.fori_loop(0, CH // L, grp_body, acc, unroll=1)

    acc = lax.fori_loop(0, NCHUNK, chunk_body, jnp.zeros((L,), jnp.float32),
                        unroll=1)
    accbuf[...] = acc
    pltpu.sync_copy(accbuf, out_hbm.at[wid])


def kernel(i, j, x, W, W_tilde, bias, bias_tilde):
    parts = _glove_sc(i.astype(jnp.int32), j.astype(jnp.int32),
                      x.astype(jnp.float32), W, W_tilde, bias, bias_tilde)
    return jnp.sum(parts) / BATCH
